# Initial kernel scaffold; baseline (speedup 1.0000x reference)
import jax
import jax.numpy as jnp
from jax.experimental import pallas as pl


def kernel(x, x0, x1, adj0, adj1, Wu, Wn, au, an, W, Wr, ar, Wl, bl):
    raise NotImplementedError("write your pallas kernel here")


# speedup vs baseline: 1.8037x; 1.8037x over previous
import functools

import jax
import jax.numpy as jnp
from jax.experimental import pallas as pl
from jax.experimental.pallas import tpu as pltpu

R = 2
H = 2
N = 4096
DIMF = 128
HID = 64
ALPHA = 0.2
NEG = -9e15

BM = 256


def _rel_body(adj_ref, xb_ref, xr_ref, xrT_ref, wua_ref, wnaT_ref,
              wncat_ref, o_ref, whn_s, sn_s):
    @pl.when(pl.program_id(0) == 0)
    def _init():
        whn_s[...] = jnp.dot(xr_ref[...], wncat_ref[...],
                             preferred_element_type=jnp.float32)
        sn_s[...] = jnp.dot(wnaT_ref[...], xrT_ref[...],
                            preferred_element_type=jnp.float32)

    su = jnp.dot(xb_ref[...], wua_ref[...], preferred_element_type=jnp.float32)
    mask = adj_ref[...] > 0
    for h in range(H):
        z = su[:, h:h + 1] + sn_s[h:h + 1, :]
        e = jnp.where(z >= 0, z, ALPHA * z)
        e = jnp.where(mask, e, NEG)
        mrow = jnp.max(e, axis=1, keepdims=True)
        w = jnp.exp(e - mrow)
        den = jnp.sum(w, axis=1, keepdims=True)
        num = jnp.dot(w, whn_s[:, h * HID:(h + 1) * HID],
                      preferred_element_type=jnp.float32)
        o = num / den
        o_ref[:, h * HID:(h + 1) * HID] = jnp.where(o > 0, o, jnp.exp(o) - 1.0)


def _rel_attention(adj, x, x_r, wua, wnaT, wncat):
    grid = (N // BM,)
    return pl.pallas_call(
        _rel_body,
        grid=grid,
        in_specs=[
            pl.BlockSpec((BM, N), lambda i: (i, 0)),
            pl.BlockSpec((BM, DIMF), lambda i: (i, 0)),
            pl.BlockSpec((N, DIMF), lambda i: (0, 0)),
            pl.BlockSpec((DIMF, N), lambda i: (0, 0)),
            pl.BlockSpec((DIMF, 8), lambda i: (0, 0)),
            pl.BlockSpec((8, DIMF), lambda i: (0, 0)),
            pl.BlockSpec((DIMF, H * HID), lambda i: (0, 0)),
        ],
        out_specs=pl.BlockSpec((BM, H * HID), lambda i: (i, 0)),
        out_shape=jax.ShapeDtypeStruct((N, H * HID), jnp.float32),
        scratch_shapes=[
            pltpu.VMEM((N, H * HID), jnp.float32),
            pltpu.VMEM((8, N), jnp.float32),
        ],
        compiler_params=pltpu.CompilerParams(
            dimension_semantics=("arbitrary",)),
    )(adj, x, x_r, x_r.T, wua, wnaT, wncat)


def _combine_body(x_ref, o0_ref, o1_ref, vxo_ref, wrwl_ref, blp_ref, out_ref):
    sx = jnp.dot(x_ref[...], vxo_ref[...],
                 preferred_element_type=jnp.float32)[:, 0:1]
    t0 = jnp.dot(o0_ref[...], vxo_ref[...],
                 preferred_element_type=jnp.float32)[:, 1:2]
    t1 = jnp.dot(o1_ref[...], vxo_ref[...],
                 preferred_element_type=jnp.float32)[:, 1:2]
    z0 = sx + t0
    z1 = sx + t1
    e0 = jnp.where(z0 >= 0, z0, ALPHA * z0)
    e1 = jnp.where(z1 >= 0, z1, ALPHA * z1)
    m = jnp.maximum(e0, e1)
    w0 = jnp.exp(e0 - m)
    w1 = jnp.exp(e1 - m)
    inv = 1.0 / (w0 + w1)
    mix = (w0 * inv) * o0_ref[...] + (w1 * inv) * o1_ref[...]
    out_ref[...] = jnp.dot(mix, wrwl_ref[...],
                           preferred_element_type=jnp.float32) + blp_ref[0:1, :]


def _combine(x, o0, o1, vxo, wrwl, blp, nclass):
    return pl.pallas_call(
        _combine_body,
        out_shape=jax.ShapeDtypeStruct((N, nclass), jnp.float32),
    )(x, o0, o1, vxo, wrwl, blp)


@functools.partial(jax.jit, static_argnums=())
def kernel(x, x0, x1, adj0, adj1, Wu, Wn, au, an, W, Wr, ar, Wl, bl):
    rhid = Wr.shape[1]
    nclass = Wl.shape[1]
    wua = jnp.einsum('rhdk,rhk->rdh', Wu, au)
    wna = jnp.einsum('rhdk,rhk->rdh', Wn, an)
    wua_p = jnp.concatenate(
        [wua, jnp.zeros((R, DIMF, 8 - H), jnp.float32)], axis=2)
    wnaT_p = jnp.concatenate(
        [jnp.swapaxes(wna, 1, 2), jnp.zeros((R, 8 - H, DIMF), jnp.float32)],
        axis=1)
    wncat = jnp.concatenate([Wn[:, h] for h in range(H)], axis=2)

    o_rel = []
    for r, (x_r, adj) in enumerate(((x0, adj0), (x1, adj1))):
        o_rel.append(_rel_attention(adj, x, x_r, wua_p[r], wnaT_p[r],
                                    wncat[r]))

    v_x = W @ (Wr @ ar[:rhid])
    v_o = Wr @ ar[rhid:]
    vxo = jnp.zeros((DIMF, 8), jnp.float32)
    vxo = vxo.at[:, 0].set(v_x).at[:, 1].set(v_o)
    wrwl = Wr @ Wl
    blp = jnp.zeros((8, nclass), jnp.float32).at[0].set(bl)
    return _combine(x, o_rel[0], o_rel[1], vxo, wrwl, blp, nclass)


# speedup vs baseline: 2.9895x; 1.6575x over previous
import functools

import jax
import jax.numpy as jnp
from jax.experimental import pallas as pl
from jax.experimental.pallas import tpu as pltpu

R = 2
H = 2
N = 4096
DIMF = 128
HID = 64
ALPHA = 0.2
NEG = -9e15

BM = 256


def _rel_body(adj_ref, xb_ref, xr_ref, xrT_ref, wua_ref, wnaT_ref,
              wncat_ref, o_ref, whn_s, sn_s):
    @pl.when(pl.program_id(0) == 0)
    def _init():
        whn = jnp.dot(xr_ref[...], wncat_ref[...],
                      preferred_element_type=jnp.float32)
        ones = jnp.ones((N, 1), jnp.float32)
        zer = jnp.zeros((N, 63), jnp.float32)
        whn_s[...] = jnp.concatenate(
            [whn[:, 0:HID], ones, zer, whn[:, HID:2 * HID], ones, zer],
            axis=1).astype(jnp.bfloat16)
        sn_s[...] = jnp.dot(wnaT_ref[...], xrT_ref[...],
                            preferred_element_type=jnp.float32
                            ).astype(jnp.bfloat16)

    su = jnp.dot(xb_ref[...], wua_ref[...],
                 preferred_element_type=jnp.float32).astype(jnp.bfloat16)
    adjw = adj_ref[...].astype(jnp.bfloat16)
    for h in range(H):
        z = su[:, h:h + 1] + sn_s[h:h + 1, :]
        e = jnp.maximum(z, jnp.bfloat16(ALPHA) * z)
        w = jnp.exp(e) * adjw
        nd = jnp.dot(w, whn_s[:, h * 128:(h + 1) * 128],
                     preferred_element_type=jnp.float32)
        o = nd[:, 0:HID] / nd[:, HID:HID + 1]
        o_ref[:, h * HID:(h + 1) * HID] = jnp.where(o > 0, o, jnp.exp(o) - 1.0)


def _rel_attention(adj, x, x_r, wua, wnaT, wncat):
    grid = (N // BM,)
    return pl.pallas_call(
        _rel_body,
        grid=grid,
        in_specs=[
            pl.BlockSpec((BM, N), lambda i: (i, 0)),
            pl.BlockSpec((BM, DIMF), lambda i: (i, 0)),
            pl.BlockSpec((N, DIMF), lambda i: (0, 0)),
            pl.BlockSpec((DIMF, N), lambda i: (0, 0)),
            pl.BlockSpec((DIMF, 8), lambda i: (0, 0)),
            pl.BlockSpec((8, DIMF), lambda i: (0, 0)),
            pl.BlockSpec((DIMF, H * HID), lambda i: (0, 0)),
        ],
        out_specs=pl.BlockSpec((BM, H * HID), lambda i: (i, 0)),
        out_shape=jax.ShapeDtypeStruct((N, H * HID), jnp.float32),
        scratch_shapes=[
            pltpu.VMEM((N, H * 128), jnp.bfloat16),
            pltpu.VMEM((8, N), jnp.bfloat16),
        ],
        compiler_params=pltpu.CompilerParams(
            dimension_semantics=("arbitrary",)),
    )(adj, x, x_r, x_r.T, wua, wnaT, wncat)


def _combine_body(x_ref, o0_ref, o1_ref, vxo_ref, wrwl_ref, blp_ref, out_ref):
    sx = jnp.dot(x_ref[...], vxo_ref[...],
                 preferred_element_type=jnp.float32)[:, 0:1]
    t0 = jnp.dot(o0_ref[...], vxo_ref[...],
                 preferred_element_type=jnp.float32)[:, 1:2]
    t1 = jnp.dot(o1_ref[...], vxo_ref[...],
                 preferred_element_type=jnp.float32)[:, 1:2]
    z0 = sx + t0
    z1 = sx + t1
    e0 = jnp.where(z0 >= 0, z0, ALPHA * z0)
    e1 = jnp.where(z1 >= 0, z1, ALPHA * z1)
    m = jnp.maximum(e0, e1)
    w0 = jnp.exp(e0 - m)
    w1 = jnp.exp(e1 - m)
    inv = 1.0 / (w0 + w1)
    mix = (w0 * inv) * o0_ref[...] + (w1 * inv) * o1_ref[...]
    out_ref[...] = jnp.dot(mix, wrwl_ref[...],
                           preferred_element_type=jnp.float32) + blp_ref[0:1, :]


def _combine(x, o0, o1, vxo, wrwl, blp, nclass):
    return pl.pallas_call(
        _combine_body,
        out_shape=jax.ShapeDtypeStruct((N, nclass), jnp.float32),
    )(x, o0, o1, vxo, wrwl, blp)


@functools.partial(jax.jit, static_argnums=())
def kernel(x, x0, x1, adj0, adj1, Wu, Wn, au, an, W, Wr, ar, Wl, bl):
    rhid = Wr.shape[1]
    nclass = Wl.shape[1]
    wua = jnp.einsum('rhdk,rhk->rdh', Wu, au)
    wna = jnp.einsum('rhdk,rhk->rdh', Wn, an)
    wua_p = jnp.concatenate(
        [wua, jnp.zeros((R, DIMF, 8 - H), jnp.float32)], axis=2)
    wnaT_p = jnp.concatenate(
        [jnp.swapaxes(wna, 1, 2), jnp.zeros((R, 8 - H, DIMF), jnp.float32)],
        axis=1)
    wncat = jnp.concatenate([Wn[:, h] for h in range(H)], axis=2)

    o_rel = []
    for r, (x_r, adj) in enumerate(((x0, adj0), (x1, adj1))):
        o_rel.append(_rel_attention(adj, x, x_r, wua_p[r], wnaT_p[r],
                                    wncat[r]))

    v_x = W @ (Wr @ ar[:rhid])
    v_o = Wr @ ar[rhid:]
    vxo = jnp.zeros((DIMF, 8), jnp.float32)
    vxo = vxo.at[:, 0].set(v_x).at[:, 1].set(v_o)
    wrwl = Wr @ Wl
    blp = jnp.zeros((8, nclass), jnp.float32).at[0].set(bl)
    return _combine(x, o_rel[0], o_rel[1], vxo, wrwl, blp, nclass)


# speedup vs baseline: 3.4697x; 1.1606x over previous
import jax
import jax.numpy as jnp
from jax.experimental import pallas as pl
from jax.experimental.pallas import tpu as pltpu

R = 2
H = 2
N = 4096
DIMF = 128
HID = 64
ALPHA = 0.2

BM = 512


def _prologue_body(x_ref, x0_ref, x1_ref, wua0_ref, wua1_ref, wna0_ref,
                   wna1_ref, wnc0_ref, wnc1_ref,
                   whn0_ref, whn1_ref, a0_ref, a1_ref, b0_ref, b1_ref):
    ones = jnp.ones((N, 1), jnp.float32)
    zer = jnp.zeros((N, 63), jnp.float32)

    def value_slab(xr, wnc):
        whn = jnp.dot(xr, wnc, preferred_element_type=jnp.float32)
        return jnp.concatenate(
            [whn[:, 0:HID], ones, zer, whn[:, HID:2 * HID], ones, zer],
            axis=1).astype(jnp.bfloat16)

    def score_exps(xv, wv):
        s = jnp.dot(xv, wv, preferred_element_type=jnp.float32)
        return jnp.concatenate(
            [jnp.exp(s[:, 0:H]), jnp.exp(ALPHA * s[:, 0:H]),
             jnp.zeros((N, 8 - 2 * H), jnp.float32)], axis=1)

    whn0_ref[...] = value_slab(x0_ref[...], wnc0_ref[...])
    whn1_ref[...] = value_slab(x1_ref[...], wnc1_ref[...])
    x = x_ref[...]
    a0_ref[...] = score_exps(x, wua0_ref[...]).astype(jnp.bfloat16)
    a1_ref[...] = score_exps(x, wua1_ref[...]).astype(jnp.bfloat16)
    b0_ref[...] = score_exps(x0_ref[...], wna0_ref[...]).T.astype(jnp.bfloat16)
    b1_ref[...] = score_exps(x1_ref[...], wna1_ref[...]).T.astype(jnp.bfloat16)


def _prologue(x, x0, x1, wua, wna, wncat):
    full = pl.BlockSpec((N, DIMF), lambda: (0, 0))
    w8 = pl.BlockSpec((DIMF, 8), lambda: (0, 0))
    wc = pl.BlockSpec((DIMF, H * HID), lambda: (0, 0))
    return pl.pallas_call(
        _prologue_body,
        in_specs=[full, full, full, w8, w8, w8, w8, wc, wc],
        out_specs=[
            pl.BlockSpec((N, H * 128), lambda: (0, 0)),
            pl.BlockSpec((N, H * 128), lambda: (0, 0)),
            pl.BlockSpec((N, 8), lambda: (0, 0)),
            pl.BlockSpec((N, 8), lambda: (0, 0)),
            pl.BlockSpec((8, N), lambda: (0, 0)),
            pl.BlockSpec((8, N), lambda: (0, 0)),
        ],
        out_shape=[
            jax.ShapeDtypeStruct((N, H * 128), jnp.bfloat16),
            jax.ShapeDtypeStruct((N, H * 128), jnp.bfloat16),
            jax.ShapeDtypeStruct((N, 8), jnp.bfloat16),
            jax.ShapeDtypeStruct((N, 8), jnp.bfloat16),
            jax.ShapeDtypeStruct((8, N), jnp.bfloat16),
            jax.ShapeDtypeStruct((8, N), jnp.bfloat16),
        ],
    )(x, x0, x1, wua[0], wua[1], wna[0], wna[1], wncat[0], wncat[1])


def _rel_body(adj_ref, a_ref, b_ref, whn_ref, o_ref):
    adjw = adj_ref[...].astype(jnp.bfloat16)
    for h in range(H):
        p1 = a_ref[:, h:h + 1] * b_ref[h:h + 1, :]
        p2 = a_ref[:, H + h:H + h + 1] * b_ref[H + h:H + h + 1, :]
        w = jnp.maximum(p1, p2) * adjw
        nd = jnp.dot(w, whn_ref[:, h * 128:(h + 1) * 128],
                     preferred_element_type=jnp.float32)
        o = nd[:, 0:HID] / nd[:, HID:HID + 1]
        o_ref[:, h * HID:(h + 1) * HID] = jnp.where(o > 0, o, jnp.exp(o) - 1.0)


def _rel_attention(adj, aexp, bexp, whn):
    return pl.pallas_call(
        _rel_body,
        grid=(N // BM,),
        in_specs=[
            pl.BlockSpec((BM, N), lambda i: (i, 0)),
            pl.BlockSpec((BM, 8), lambda i: (i, 0)),
            pl.BlockSpec((8, N), lambda i: (0, 0)),
            pl.BlockSpec((N, H * 128), lambda i: (0, 0)),
        ],
        out_specs=pl.BlockSpec((BM, H * HID), lambda i: (i, 0)),
        out_shape=jax.ShapeDtypeStruct((N, H * HID), jnp.float32),
        compiler_params=pltpu.CompilerParams(
            dimension_semantics=("arbitrary",)),
    )(adj, aexp, bexp, whn)


def _combine_body(x_ref, o0_ref, o1_ref, vxo_ref, wrwl_ref, blp_ref, out_ref):
    sx = jnp.dot(x_ref[...], vxo_ref[...],
                 preferred_element_type=jnp.float32)[:, 0:1]
    t0 = jnp.dot(o0_ref[...], vxo_ref[...],
                 preferred_element_type=jnp.float32)[:, 1:2]
    t1 = jnp.dot(o1_ref[...], vxo_ref[...],
                 preferred_element_type=jnp.float32)[:, 1:2]
    z0 = sx + t0
    z1 = sx + t1
    e0 = jnp.where(z0 >= 0, z0, ALPHA * z0)
    e1 = jnp.where(z1 >= 0, z1, ALPHA * z1)
    m = jnp.maximum(e0, e1)
    w0 = jnp.exp(e0 - m)
    w1 = jnp.exp(e1 - m)
    inv = 1.0 / (w0 + w1)
    mix = (w0 * inv) * o0_ref[...] + (w1 * inv) * o1_ref[...]
    out_ref[...] = jnp.dot(mix, wrwl_ref[...],
                           preferred_element_type=jnp.float32) + blp_ref[0:1, :]


def _combine(x, o0, o1, vxo, wrwl, blp, nclass):
    return pl.pallas_call(
        _combine_body,
        out_shape=jax.ShapeDtypeStruct((N, nclass), jnp.float32),
    )(x, o0, o1, vxo, wrwl, blp)


def kernel(x, x0, x1, adj0, adj1, Wu, Wn, au, an, W, Wr, ar, Wl, bl):
    rhid = Wr.shape[1]
    nclass = Wl.shape[1]
    wua = jnp.einsum('rhdk,rhk->rdh', Wu, au)
    wna = jnp.einsum('rhdk,rhk->rdh', Wn, an)
    wua_p = jnp.concatenate(
        [wua, jnp.zeros((R, DIMF, 8 - H), jnp.float32)], axis=2)
    wna_p = jnp.concatenate(
        [wna, jnp.zeros((R, DIMF, 8 - H), jnp.float32)], axis=2)
    wncat = jnp.concatenate([Wn[:, h] for h in range(H)], axis=2)

    whn0, whn1, a0, a1, b0, b1 = _prologue(x, x0, x1, wua_p, wna_p, wncat)
    o0 = _rel_attention(adj0, a0, b0, whn0)
    o1 = _rel_attention(adj1, a1, b1, whn1)

    v_x = W @ (Wr @ ar[:rhid])
    v_o = Wr @ ar[rhid:]
    vxo = jnp.zeros((DIMF, 8), jnp.float32)
    vxo = vxo.at[:, 0].set(v_x).at[:, 1].set(v_o)
    wrwl = Wr @ Wl
    blp = jnp.zeros((8, nclass), jnp.float32).at[0].set(bl)
    return _combine(x, o0, o1, vxo, wrwl, blp, nclass)


# speedup vs baseline: 3.5642x; 1.0272x over previous
import jax
import jax.numpy as jnp
from jax.experimental import pallas as pl
from jax.experimental.pallas import tpu as pltpu

R = 2
H = 2
N = 4096
DIMF = 128
HID = 64
ALPHA = 0.2

BM = 512


def _prologue_body(x_ref, x0_ref, x1_ref, wua0_ref, wua1_ref, wna0_ref,
                   wna1_ref, wnc0_ref, wnc1_ref,
                   whn0_ref, whn1_ref, a0_ref, a1_ref, b0_ref, b1_ref):
    ones = jnp.ones((N, 1), jnp.float32)
    zer = jnp.zeros((N, 63), jnp.float32)

    def value_slab(xr, wnc):
        whn = jnp.dot(xr, wnc, preferred_element_type=jnp.float32)
        return jnp.concatenate(
            [whn[:, 0:HID], ones, zer, whn[:, HID:2 * HID], ones, zer],
            axis=1).astype(jnp.bfloat16)

    def score_exps_dst(xv, wv):
        s = jnp.dot(xv, wv, preferred_element_type=jnp.float32)
        return jnp.concatenate(
            [jnp.exp((1.0 - ALPHA) * s[:, 0:H]),
             jnp.zeros((N, 8 - H), jnp.float32)], axis=1)

    def score_exps_src(xv, wv):
        s = jnp.dot(xv, wv, preferred_element_type=jnp.float32)
        return jnp.concatenate(
            [jnp.exp(s[:, 0:H]), jnp.exp(ALPHA * s[:, 0:H]),
             jnp.zeros((N, 8 - 2 * H), jnp.float32)], axis=1)

    whn0_ref[...] = value_slab(x0_ref[...], wnc0_ref[...])
    whn1_ref[...] = value_slab(x1_ref[...], wnc1_ref[...])
    x = x_ref[...]
    a0_ref[...] = score_exps_dst(x, wua0_ref[...]).astype(jnp.bfloat16)
    a1_ref[...] = score_exps_dst(x, wua1_ref[...]).astype(jnp.bfloat16)
    b0_ref[...] = score_exps_src(x0_ref[...], wna0_ref[...]).T.astype(jnp.bfloat16)
    b1_ref[...] = score_exps_src(x1_ref[...], wna1_ref[...]).T.astype(jnp.bfloat16)


def _prologue(x, x0, x1, wua, wna, wncat):
    full = pl.BlockSpec((N, DIMF), lambda: (0, 0))
    w8 = pl.BlockSpec((DIMF, 8), lambda: (0, 0))
    wc = pl.BlockSpec((DIMF, H * HID), lambda: (0, 0))
    return pl.pallas_call(
        _prologue_body,
        in_specs=[full, full, full, w8, w8, w8, w8, wc, wc],
        out_specs=[
            pl.BlockSpec((N, H * 128), lambda: (0, 0)),
            pl.BlockSpec((N, H * 128), lambda: (0, 0)),
            pl.BlockSpec((N, 8), lambda: (0, 0)),
            pl.BlockSpec((N, 8), lambda: (0, 0)),
            pl.BlockSpec((8, N), lambda: (0, 0)),
            pl.BlockSpec((8, N), lambda: (0, 0)),
        ],
        out_shape=[
            jax.ShapeDtypeStruct((N, H * 128), jnp.bfloat16),
            jax.ShapeDtypeStruct((N, H * 128), jnp.bfloat16),
            jax.ShapeDtypeStruct((N, 8), jnp.bfloat16),
            jax.ShapeDtypeStruct((N, 8), jnp.bfloat16),
            jax.ShapeDtypeStruct((8, N), jnp.bfloat16),
            jax.ShapeDtypeStruct((8, N), jnp.bfloat16),
        ],
    )(x, x0, x1, wua[0], wua[1], wna[0], wna[1], wncat[0], wncat[1])


def _rel_body(adj_ref, a_ref, b_ref, whn_ref, o_ref):
    adjw = adj_ref[...].astype(jnp.bfloat16)
    for h in range(H):
        q1 = a_ref[:, h:h + 1] * b_ref[h:h + 1, :]
        w = jnp.maximum(q1, b_ref[H + h:H + h + 1, :]) * adjw
        nd = jnp.dot(w, whn_ref[:, h * 128:(h + 1) * 128],
                     preferred_element_type=jnp.float32)
        o = nd[:, 0:HID] / nd[:, HID:HID + 1]
        o_ref[:, h * HID:(h + 1) * HID] = jnp.where(o > 0, o, jnp.exp(o) - 1.0)


def _rel_attention(adj, aexp, bexp, whn):
    return pl.pallas_call(
        _rel_body,
        grid=(N // BM,),
        in_specs=[
            pl.BlockSpec((BM, N), lambda i: (i, 0)),
            pl.BlockSpec((BM, 8), lambda i: (i, 0)),
            pl.BlockSpec((8, N), lambda i: (0, 0)),
            pl.BlockSpec((N, H * 128), lambda i: (0, 0)),
        ],
        out_specs=pl.BlockSpec((BM, H * HID), lambda i: (i, 0)),
        out_shape=jax.ShapeDtypeStruct((N, H * HID), jnp.float32),
        compiler_params=pltpu.CompilerParams(
            dimension_semantics=("parallel",)),
    )(adj, aexp, bexp, whn)


def _combine_body(x_ref, o0_ref, o1_ref, vxo_ref, wrwl_ref, blp_ref, out_ref):
    sx = jnp.dot(x_ref[...], vxo_ref[...],
                 preferred_element_type=jnp.float32)[:, 0:1]
    t0 = jnp.dot(o0_ref[...], vxo_ref[...],
                 preferred_element_type=jnp.float32)[:, 1:2]
    t1 = jnp.dot(o1_ref[...], vxo_ref[...],
                 preferred_element_type=jnp.float32)[:, 1:2]
    z0 = sx + t0
    z1 = sx + t1
    e0 = jnp.where(z0 >= 0, z0, ALPHA * z0)
    e1 = jnp.where(z1 >= 0, z1, ALPHA * z1)
    m = jnp.maximum(e0, e1)
    w0 = jnp.exp(e0 - m)
    w1 = jnp.exp(e1 - m)
    inv = 1.0 / (w0 + w1)
    mix = (w0 * inv) * o0_ref[...] + (w1 * inv) * o1_ref[...]
    out_ref[...] = jnp.dot(mix, wrwl_ref[...],
                           preferred_element_type=jnp.float32) + blp_ref[0:1, :]


def _combine(x, o0, o1, vxo, wrwl, blp, nclass):
    return pl.pallas_call(
        _combine_body,
        out_shape=jax.ShapeDtypeStruct((N, nclass), jnp.float32),
    )(x, o0, o1, vxo, wrwl, blp)


def kernel(x, x0, x1, adj0, adj1, Wu, Wn, au, an, W, Wr, ar, Wl, bl):
    rhid = Wr.shape[1]
    nclass = Wl.shape[1]
    wua = jnp.einsum('rhdk,rhk->rdh', Wu, au)
    wna = jnp.einsum('rhdk,rhk->rdh', Wn, an)
    wua_p = jnp.concatenate(
        [wua, jnp.zeros((R, DIMF, 8 - H), jnp.float32)], axis=2)
    wna_p = jnp.concatenate(
        [wna, jnp.zeros((R, DIMF, 8 - H), jnp.float32)], axis=2)
    wncat = jnp.concatenate([Wn[:, h] for h in range(H)], axis=2)

    whn0, whn1, a0, a1, b0, b1 = _prologue(x, x0, x1, wua_p, wna_p, wncat)
    o0 = _rel_attention(adj0, a0, b0, whn0)
    o1 = _rel_attention(adj1, a1, b1, whn1)

    v_x = W @ (Wr @ ar[:rhid])
    v_o = Wr @ ar[rhid:]
    vxo = jnp.zeros((DIMF, 8), jnp.float32)
    vxo = vxo.at[:, 0].set(v_x).at[:, 1].set(v_o)
    wrwl = Wr @ Wl
    blp = jnp.zeros((8, nclass), jnp.float32).at[0].set(bl)
    return _combine(x, o0, o1, vxo, wrwl, blp, nclass)


# speedup vs baseline: 3.6380x; 1.0207x over previous
import jax
import jax.numpy as jnp
from jax.experimental import pallas as pl
from jax.experimental.pallas import tpu as pltpu

R = 2
H = 2
N = 4096
DIMF = 128
HID = 64
ALPHA = 0.2

BM = 512


def _prologue_compute(x, xr, wua, wna, wnc, whn_s, a_s, b_s):
    ones = jnp.ones((N, 1), jnp.float32)
    zer = jnp.zeros((N, 63), jnp.float32)
    whn = jnp.dot(xr, wnc, preferred_element_type=jnp.float32)
    whn_s[...] = jnp.concatenate(
        [whn[:, 0:HID], ones, zer, whn[:, HID:2 * HID], ones, zer],
        axis=1).astype(jnp.bfloat16)
    su = jnp.dot(x, wua, preferred_element_type=jnp.float32)
    a_s[...] = jnp.concatenate(
        [jnp.exp((1.0 - ALPHA) * su[:, 0:H]),
         jnp.zeros((N, 8 - H), jnp.float32)], axis=1).astype(jnp.bfloat16)
    sn = jnp.dot(xr, wna, preferred_element_type=jnp.float32)
    b_s[...] = jnp.concatenate(
        [jnp.exp(sn[:, 0:H]), jnp.exp(ALPHA * sn[:, 0:H]),
         jnp.zeros((N, 8 - 2 * H), jnp.float32)],
        axis=1).T.astype(jnp.bfloat16)


def _att_block(adj_ref, whn_s, a_s, b_s, i):
    adjw = adj_ref[...].astype(jnp.bfloat16)
    a = a_s[pl.ds(i * BM, BM), :]
    outs = []
    for h in range(H):
        q1 = a[:, h:h + 1] * b_s[h:h + 1, :]
        w = jnp.maximum(q1, b_s[H + h:H + h + 1, :]) * adjw
        nd = jnp.dot(w, whn_s[:, h * 128:(h + 1) * 128],
                     preferred_element_type=jnp.float32)
        o = nd[:, 0:HID] / nd[:, HID:HID + 1]
        outs.append(jnp.where(o > 0, o, jnp.exp(o) - 1.0))
    return jnp.concatenate(outs, axis=1)


def _rel0_body(x_ref, xr_ref, wua_ref, wna_ref, wnc_ref, adj_ref,
               o_ref, whn_s, a_s, b_s):
    i = pl.program_id(0)

    @pl.when(i == 0)
    def _():
        _prologue_compute(x_ref[...], xr_ref[...], wua_ref[...],
                          wna_ref[...], wnc_ref[...], whn_s, a_s, b_s)

    o_ref[...] = _att_block(adj_ref, whn_s, a_s, b_s, i)


def _rel1_body(x_ref, xr_ref, wua_ref, wna_ref, wnc_ref, adj_ref,
               o0_ref, xb_ref, vxo_ref, wrwl_ref, blp_ref,
               out_ref, whn_s, a_s, b_s):
    i = pl.program_id(0)

    @pl.when(i == 0)
    def _():
        _prologue_compute(x_ref[...], xr_ref[...], wua_ref[...],
                          wna_ref[...], wnc_ref[...], whn_s, a_s, b_s)

    o1 = _att_block(adj_ref, whn_s, a_s, b_s, i)
    o0 = o0_ref[...]
    sx = jnp.dot(xb_ref[...], vxo_ref[...],
                 preferred_element_type=jnp.float32)[:, 0:1]
    t0 = jnp.dot(o0, vxo_ref[...],
                 preferred_element_type=jnp.float32)[:, 1:2]
    t1 = jnp.dot(o1, vxo_ref[...],
                 preferred_element_type=jnp.float32)[:, 1:2]
    z0 = sx + t0
    z1 = sx + t1
    e0 = jnp.where(z0 >= 0, z0, ALPHA * z0)
    e1 = jnp.where(z1 >= 0, z1, ALPHA * z1)
    m = jnp.maximum(e0, e1)
    w0 = jnp.exp(e0 - m)
    w1 = jnp.exp(e1 - m)
    inv = 1.0 / (w0 + w1)
    mix = (w0 * inv) * o0 + (w1 * inv) * o1
    out_ref[...] = jnp.dot(mix, wrwl_ref[...],
                           preferred_element_type=jnp.float32) + blp_ref[0:1, :]


_FULL = pl.BlockSpec((N, DIMF), lambda i: (0, 0))
_W8 = pl.BlockSpec((DIMF, 8), lambda i: (0, 0))
_WC = pl.BlockSpec((DIMF, H * HID), lambda i: (0, 0))
_SCRATCH = [
    pltpu.VMEM((N, H * 128), jnp.bfloat16),
    pltpu.VMEM((N, 8), jnp.bfloat16),
    pltpu.VMEM((8, N), jnp.bfloat16),
]


def _rel0(x, xr, wua, wna, wnc, adj):
    return pl.pallas_call(
        _rel0_body,
        grid=(N // BM,),
        in_specs=[_FULL, _FULL, _W8, _W8, _WC,
                  pl.BlockSpec((BM, N), lambda i: (i, 0))],
        out_specs=pl.BlockSpec((BM, H * HID), lambda i: (i, 0)),
        out_shape=jax.ShapeDtypeStruct((N, H * HID), jnp.float32),
        scratch_shapes=_SCRATCH,
        compiler_params=pltpu.CompilerParams(
            dimension_semantics=("arbitrary",)),
    )(x, xr, wua, wna, wnc, adj)


def _rel1(x, xr, wua, wna, wnc, adj, o0, vxo, wrwl, blp, nclass):
    return pl.pallas_call(
        _rel1_body,
        grid=(N // BM,),
        in_specs=[_FULL, _FULL, _W8, _W8, _WC,
                  pl.BlockSpec((BM, N), lambda i: (i, 0)),
                  pl.BlockSpec((BM, H * HID), lambda i: (i, 0)),
                  pl.BlockSpec((BM, DIMF), lambda i: (i, 0)),
                  _W8,
                  pl.BlockSpec((H * HID, 8), lambda i: (0, 0)),
                  pl.BlockSpec((8, 8), lambda i: (0, 0))],
        out_specs=pl.BlockSpec((BM, 8), lambda i: (i, 0)),
        out_shape=jax.ShapeDtypeStruct((N, 8), jnp.float32),
        scratch_shapes=_SCRATCH,
        compiler_params=pltpu.CompilerParams(
            dimension_semantics=("arbitrary",)),
    )(x, xr, wua, wna, wnc, adj, o0, x, vxo, wrwl, blp)


def kernel(x, x0, x1, adj0, adj1, Wu, Wn, au, an, W, Wr, ar, Wl, bl):
    rhid = Wr.shape[1]
    nclass = Wl.shape[1]
    wua = jnp.einsum('rhdk,rhk->rdh', Wu, au)
    wna = jnp.einsum('rhdk,rhk->rdh', Wn, an)
    wua_p = jnp.concatenate(
        [wua, jnp.zeros((R, DIMF, 8 - H), jnp.float32)], axis=2)
    wna_p = jnp.concatenate(
        [wna, jnp.zeros((R, DIMF, 8 - H), jnp.float32)], axis=2)
    wncat = jnp.concatenate([Wn[:, h] for h in range(H)], axis=2)

    v_x = W @ (Wr @ ar[:rhid])
    v_o = Wr @ ar[rhid:]
    vxo = jnp.zeros((DIMF, 8), jnp.float32)
    vxo = vxo.at[:, 0].set(v_x).at[:, 1].set(v_o)
    wrwl = Wr @ Wl
    blp = jnp.zeros((8, nclass), jnp.float32).at[0].set(bl)

    o0 = _rel0(x, x0, wua_p[0], wna_p[0], wncat[0], adj0)
    out = _rel1(x, x1, wua_p[1], wna_p[1], wncat[1], adj1,
                o0, vxo, wrwl, blp, nclass)
    return out


# speedup vs baseline: 4.0458x; 1.1121x over previous
import jax
import jax.numpy as jnp
from jax.experimental import pallas as pl
from jax.experimental.pallas import tpu as pltpu

R = 2
H = 2
N = 4096
DIMF = 128
HID = 64
ALPHA = 0.2

BM = 512


def _prologue_compute(x, xr, wua, wna, wnc, whn_s, a_s, b_s):
    ones = jnp.ones((N, 1), jnp.float32)
    zer = jnp.zeros((N, 63), jnp.float32)
    whn = jnp.dot(xr, wnc, preferred_element_type=jnp.float32)
    whn_s[...] = jnp.concatenate(
        [whn[:, 0:HID], ones, zer, whn[:, HID:2 * HID], ones, zer],
        axis=1).astype(jnp.bfloat16)
    su = jnp.dot(x, wua, preferred_element_type=jnp.float32)
    a_s[...] = jnp.concatenate(
        [jnp.exp((1.0 - ALPHA) * su[:, 0:H]),
         jnp.zeros((N, 8 - H), jnp.float32)], axis=1).astype(jnp.bfloat16)
    sn = jnp.dot(xr, wna, preferred_element_type=jnp.float32)
    b_s[...] = jnp.concatenate(
        [jnp.exp(sn[:, 0:H]), jnp.exp(ALPHA * sn[:, 0:H]),
         jnp.zeros((N, 8 - 2 * H), jnp.float32)],
        axis=1).T.astype(jnp.bfloat16)


def _att_block(adj_ref, whn_s, a_s, b_s, i):
    adjw = adj_ref[...].astype(jnp.bfloat16)
    a = a_s[pl.ds(i * BM, BM), :]
    outs = []
    for h in range(H):
        q1 = a[:, h:h + 1] * b_s[h:h + 1, :]
        w = jnp.maximum(q1, b_s[H + h:H + h + 1, :]) * adjw
        nd = jnp.dot(w, whn_s[:, h * 128:(h + 1) * 128],
                     preferred_element_type=jnp.float32)
        o = nd[:, 0:HID] / nd[:, HID:HID + 1]
        outs.append(jnp.where(o > 0, o, jnp.exp(o) - 1.0))
    return jnp.concatenate(outs, axis=1)


def _body(x_ref, x0_ref, x1_ref, wua0_ref, wna0_ref, wnc0_ref,
          wua1_ref, wna1_ref, wnc1_ref, adj0_ref, adj1_ref,
          xb_ref, vxo_ref, wrwl_ref, blp_ref, out_ref,
          whn0_s, a0_s, b0_s, whn1_s, a1_s, b1_s):
    i = pl.program_id(0)

    @pl.when(i == 0)
    def _():
        _prologue_compute(x_ref[...], x0_ref[...], wua0_ref[...],
                          wna0_ref[...], wnc0_ref[...], whn0_s, a0_s, b0_s)
        _prologue_compute(x_ref[...], x1_ref[...], wua1_ref[...],
                          wna1_ref[...], wnc1_ref[...], whn1_s, a1_s, b1_s)

    o0 = _att_block(adj0_ref, whn0_s, a0_s, b0_s, i)
    o1 = _att_block(adj1_ref, whn1_s, a1_s, b1_s, i)

    sx = jnp.dot(xb_ref[...], vxo_ref[...],
                 preferred_element_type=jnp.float32)[:, 0:1]
    t0 = jnp.dot(o0, vxo_ref[...],
                 preferred_element_type=jnp.float32)[:, 1:2]
    t1 = jnp.dot(o1, vxo_ref[...],
                 preferred_element_type=jnp.float32)[:, 1:2]
    z0 = sx + t0
    z1 = sx + t1
    e0 = jnp.where(z0 >= 0, z0, ALPHA * z0)
    e1 = jnp.where(z1 >= 0, z1, ALPHA * z1)
    m = jnp.maximum(e0, e1)
    w0 = jnp.exp(e0 - m)
    w1 = jnp.exp(e1 - m)
    inv = 1.0 / (w0 + w1)
    mix = (w0 * inv) * o0 + (w1 * inv) * o1
    out_ref[...] = jnp.dot(mix, wrwl_ref[...],
                           preferred_element_type=jnp.float32) + blp_ref[0:1, :]


def kernel(x, x0, x1, adj0, adj1, Wu, Wn, au, an, W, Wr, ar, Wl, bl):
    rhid = Wr.shape[1]
    nclass = Wl.shape[1]
    wua = jnp.einsum('rhdk,rhk->rdh', Wu, au)
    wna = jnp.einsum('rhdk,rhk->rdh', Wn, an)
    wua_p = jnp.concatenate(
        [wua, jnp.zeros((R, DIMF, 8 - H), jnp.float32)], axis=2)
    wna_p = jnp.concatenate(
        [wna, jnp.zeros((R, DIMF, 8 - H), jnp.float32)], axis=2)
    wncat = jnp.concatenate([Wn[:, h] for h in range(H)], axis=2)

    v_x = W @ (Wr @ ar[:rhid])
    v_o = Wr @ ar[rhid:]
    vxo = jnp.zeros((DIMF, 8), jnp.float32)
    vxo = vxo.at[:, 0].set(v_x).at[:, 1].set(v_o)
    wrwl = Wr @ Wl
    blp = jnp.zeros((8, nclass), jnp.float32).at[0].set(bl)

    full = pl.BlockSpec((N, DIMF), lambda i: (0, 0))
    w8 = pl.BlockSpec((DIMF, 8), lambda i: (0, 0))
    wc = pl.BlockSpec((DIMF, H * HID), lambda i: (0, 0))
    adjb = pl.BlockSpec((BM, N), lambda i: (i, 0))
    scratch = [
        pltpu.VMEM((N, H * 128), jnp.bfloat16),
        pltpu.VMEM((N, 8), jnp.bfloat16),
        pltpu.VMEM((8, N), jnp.bfloat16),
    ] * R

    return pl.pallas_call(
        _body,
        grid=(N // BM,),
        in_specs=[full, full, full, w8, w8, wc, w8, w8, wc,
                  adjb, adjb,
                  pl.BlockSpec((BM, DIMF), lambda i: (i, 0)),
                  w8,
                  pl.BlockSpec((H * HID, 8), lambda i: (0, 0)),
                  pl.BlockSpec((8, 8), lambda i: (0, 0))],
        out_specs=pl.BlockSpec((BM, 8), lambda i: (i, 0)),
        out_shape=jax.ShapeDtypeStruct((N, 8), jnp.float32),
        scratch_shapes=scratch,
        compiler_params=pltpu.CompilerParams(
            dimension_semantics=("arbitrary",)),
    )(x, x0, x1, wua_p[0], wna_p[0], wncat[0], wua_p[1], wna_p[1], wncat[1],
      adj0, adj1, x, vxo, wrwl, blp)


# speedup vs baseline: 4.0655x; 1.0049x over previous
import jax
import jax.numpy as jnp
from jax.experimental import pallas as pl
from jax.experimental.pallas import tpu as pltpu

R = 2
H = 2
N = 4096
DIMF = 128
HID = 64
ALPHA = 0.2

BM = 512


def _prologue_compute(x, xr, wua, wna, wnc, whn_s, a_s, b_s):
    ones = jnp.ones((N, 1), jnp.float32)
    zer = jnp.zeros((N, 63), jnp.float32)
    whn = jnp.dot(xr, wnc, preferred_element_type=jnp.float32)
    whn_s[...] = jnp.concatenate(
        [whn[:, 0:HID], ones, zer, whn[:, HID:2 * HID], ones, zer],
        axis=1).astype(jnp.bfloat16)
    su = jnp.dot(x, wua, preferred_element_type=jnp.float32)
    a_s[...] = jnp.concatenate(
        [jnp.exp((1.0 - ALPHA) * su[:, 0:H]),
         jnp.zeros((N, 8 - H), jnp.float32)], axis=1).astype(jnp.bfloat16)
    sn = jnp.dot(xr, wna, preferred_element_type=jnp.float32)
    b_s[...] = jnp.concatenate(
        [jnp.exp(sn[:, 0:H]), jnp.exp(ALPHA * sn[:, 0:H]),
         jnp.zeros((N, 8 - 2 * H), jnp.float32)],
        axis=1).T.astype(jnp.bfloat16)


def _att_block(adj_refs, whn_s, a_s, b_s, i):
    nh = len(adj_refs)
    cw = N // nh
    a = a_s[pl.ds(i * BM, BM), :]
    nds = []
    for h in range(H):
        nd = jnp.zeros((BM, 128), jnp.float32)
        for j, adj_ref in enumerate(adj_refs):
            adjw = adj_ref[...].astype(jnp.bfloat16)
            q1 = a[:, h:h + 1] * b_s[h:h + 1, pl.ds(j * cw, cw)]
            w = jnp.maximum(q1, b_s[H + h:H + h + 1, pl.ds(j * cw, cw)]) * adjw
            nd = nd + jnp.dot(w, whn_s[pl.ds(j * cw, cw),
                                       h * 128:(h + 1) * 128],
                              preferred_element_type=jnp.float32)
        nds.append(nd)
    outs = []
    for h in range(H):
        o = nds[h][:, 0:HID] / nds[h][:, HID:HID + 1]
        outs.append(jnp.where(o > 0, o, jnp.exp(o) - 1.0))
    return jnp.concatenate(outs, axis=1)


def _body(x_ref, x0_ref, x1_ref, wua0_ref, wna0_ref, wnc0_ref,
          wua1_ref, wna1_ref, wnc1_ref, adj0l_ref, adj0r_ref,
          adj1l_ref, adj1r_ref,
          xb_ref, vxo_ref, wrwl_ref, blp_ref, out_ref,
          whn0_s, a0_s, b0_s, whn1_s, a1_s, b1_s):
    i = pl.program_id(0)

    @pl.when(i == 0)
    def _():
        _prologue_compute(x_ref[...], x0_ref[...], wua0_ref[...],
                          wna0_ref[...], wnc0_ref[...], whn0_s, a0_s, b0_s)
        _prologue_compute(x_ref[...], x1_ref[...], wua1_ref[...],
                          wna1_ref[...], wnc1_ref[...], whn1_s, a1_s, b1_s)

    o0 = _att_block([adj0l_ref, adj0r_ref], whn0_s, a0_s, b0_s, i)
    o1 = _att_block([adj1l_ref, adj1r_ref], whn1_s, a1_s, b1_s, i)

    sx = jnp.dot(xb_ref[...], vxo_ref[...],
                 preferred_element_type=jnp.float32)[:, 0:1]
    t0 = jnp.dot(o0, vxo_ref[...],
                 preferred_element_type=jnp.float32)[:, 1:2]
    t1 = jnp.dot(o1, vxo_ref[...],
                 preferred_element_type=jnp.float32)[:, 1:2]
    z0 = sx + t0
    z1 = sx + t1
    e0 = jnp.where(z0 >= 0, z0, ALPHA * z0)
    e1 = jnp.where(z1 >= 0, z1, ALPHA * z1)
    m = jnp.maximum(e0, e1)
    w0 = jnp.exp(e0 - m)
    w1 = jnp.exp(e1 - m)
    inv = 1.0 / (w0 + w1)
    mix = (w0 * inv) * o0 + (w1 * inv) * o1
    out_ref[...] = jnp.dot(mix, wrwl_ref[...],
                           preferred_element_type=jnp.float32) + blp_ref[0:1, :]


def kernel(x, x0, x1, adj0, adj1, Wu, Wn, au, an, W, Wr, ar, Wl, bl):
    rhid = Wr.shape[1]
    nclass = Wl.shape[1]
    wua = jnp.einsum('rhdk,rhk->rdh', Wu, au)
    wna = jnp.einsum('rhdk,rhk->rdh', Wn, an)
    wua_p = jnp.concatenate(
        [wua, jnp.zeros((R, DIMF, 8 - H), jnp.float32)], axis=2)
    wna_p = jnp.concatenate(
        [wna, jnp.zeros((R, DIMF, 8 - H), jnp.float32)], axis=2)
    wncat = jnp.concatenate([Wn[:, h] for h in range(H)], axis=2)

    v_x = W @ (Wr @ ar[:rhid])
    v_o = Wr @ ar[rhid:]
    vxo = jnp.zeros((DIMF, 8), jnp.float32)
    vxo = vxo.at[:, 0].set(v_x).at[:, 1].set(v_o)
    wrwl = Wr @ Wl
    blp = jnp.zeros((8, nclass), jnp.float32).at[0].set(bl)

    full = pl.BlockSpec((N, DIMF), lambda i: (0, 0))
    w8 = pl.BlockSpec((DIMF, 8), lambda i: (0, 0))
    wc = pl.BlockSpec((DIMF, H * HID), lambda i: (0, 0))
    adjl = pl.BlockSpec((BM, N // 2), lambda i: (i, 0))
    adjr = pl.BlockSpec((BM, N // 2), lambda i: (i, 1))
    scratch = [
        pltpu.VMEM((N, H * 128), jnp.bfloat16),
        pltpu.VMEM((N, 8), jnp.bfloat16),
        pltpu.VMEM((8, N), jnp.bfloat16),
    ] * R

    return pl.pallas_call(
        _body,
        grid=(N // BM,),
        in_specs=[full, full, full, w8, w8, wc, w8, w8, wc,
                  adjl, adjr, adjl, adjr,
                  pl.BlockSpec((BM, DIMF), lambda i: (i, 0)),
                  w8,
                  pl.BlockSpec((H * HID, 8), lambda i: (0, 0)),
                  pl.BlockSpec((8, 8), lambda i: (0, 0))],
        out_specs=pl.BlockSpec((BM, 8), lambda i: (i, 0)),
        out_shape=jax.ShapeDtypeStruct((N, 8), jnp.float32),
        scratch_shapes=scratch,
        compiler_params=pltpu.CompilerParams(
            dimension_semantics=("arbitrary",)),
    )(x, x0, x1, wua_p[0], wna_p[0], wncat[0], wua_p[1], wna_p[1], wncat[1],
      adj0, adj0, adj1, adj1, x, vxo, wrwl, blp)
